# Initial kernel scaffold; baseline (speedup 1.0000x reference)
import jax
import jax.numpy as jnp
from jax.experimental import pallas as pl


def kernel(x, table, W, b):
    raise NotImplementedError("write your pallas kernel here")


# speedup vs baseline: 11.6875x; 11.6875x over previous
import functools

import jax
import jax.numpy as jnp
from jax import lax
from jax.experimental import pallas as pl
from jax.experimental.pallas import tpu as pltpu
from jax.experimental.pallas import tpu_sc as plsc

VOCAB = 1000000
EMBED = 32
BATCH = 16384
HIST = 200


_S_BLK = 8192
_S_GRID = (VOCAB + _S_BLK - 1) // _S_BLK


def _s_body(t_ref, w_ref, o_ref):
    o_ref[...] = jnp.sum(t_ref[...] * w_ref[...], axis=1)


def _compute_s(table, w_scaled):
    return pl.pallas_call(
        _s_body,
        grid=(_S_GRID,),
        in_specs=[
            pl.BlockSpec((_S_BLK, EMBED), lambda i: (i, 0)),
            pl.BlockSpec((1, EMBED), lambda i: (0, 0)),
        ],
        out_specs=pl.BlockSpec((_S_BLK,), lambda i: (i,)),
        out_shape=jax.ShapeDtypeStruct((VOCAB,), jnp.float32),
    )(table, w_scaled)


_NC = 2
_NS = 16
_NW = _NC * _NS
_ROWS_W = BATCH // _NW
_GRP = 16
_NGRP = _ROWS_W // _GRP

_GSZ = HIST * _GRP


@functools.cache
def _make_sc_pool():
    mesh = plsc.VectorSubcoreMesh(core_axis_name="c", subcore_axis_name="s")

    @functools.partial(
        pl.kernel,
        out_type=jax.ShapeDtypeStruct((BATCH,), jnp.float32),
        mesh=mesh,
        scratch_types=[
            pltpu.VMEM((_GSZ,), jnp.int32),
            pltpu.VMEM((_GSZ,), jnp.float32),
            pltpu.VMEM((_GRP,), jnp.float32),
            pltpu.SemaphoreType.DMA,
        ],
    )
    def _sc_pool(xg_hbm, s_hbm, bv_hbm, out_hbm, idx_v, vals_v, out_v, sem):
        wid = lax.axis_index("s") * _NC + lax.axis_index("c")
        gbase = wid * _NGRP
        pltpu.sync_copy(bv_hbm, out_v)
        bv = out_v[...]

        def group(g, carry):
            ggl = gbase + g
            pltpu.sync_copy(xg_hbm.at[pl.ds(ggl * _GSZ, _GSZ)], idx_v)
            pltpu.async_copy(s_hbm.at[idx_v], vals_v, sem).wait()

            def accum(l, a):
                return a + vals_v[pl.ds(l * _GRP, _GRP)]

            acc = lax.fori_loop(0, HIST, accum, bv)
            out_v[...] = 1.0 / (1.0 + jnp.exp(-acc))
            pltpu.sync_copy(out_v, out_hbm.at[pl.ds(ggl * _GRP, _GRP)])
            return carry

        lax.fori_loop(0, _NGRP, group, 0)

    return _sc_pool


def kernel(x, table, W, b):
    w_scaled = (W / HIST).astype(jnp.float32)
    s = _compute_s(table, w_scaled)
    xg = (
        x.astype(jnp.int32)
        .reshape(BATCH // _GRP, _GRP, HIST)
        .swapaxes(1, 2)
        .reshape(BATCH // _GRP * _GSZ)
    )
    bv = jnp.broadcast_to(b.astype(jnp.float32), (_GRP,))
    return _make_sc_pool()(xg, s, bv)


# speedup vs baseline: 15.5417x; 1.3298x over previous
import functools

import jax
import jax.numpy as jnp
from jax import lax
from jax.experimental import pallas as pl
from jax.experimental.pallas import tpu as pltpu
from jax.experimental.pallas import tpu_sc as plsc

VOCAB = 1000000
EMBED = 32
BATCH = 16384
HIST = 200


_S_BLK = 8192
_S_GRID = (VOCAB + _S_BLK - 1) // _S_BLK


def _s_body(w_ref, t_ref, o_ref):
    o = lax.dot_general(
        w_ref[...], t_ref[...],
        dimension_numbers=(((1,), (1,)), ((), ())),
        preferred_element_type=jnp.float32,
    )
    o_ref[...] = o[0]


def _compute_s(table, w_scaled):
    w8 = jnp.broadcast_to(w_scaled, (8, EMBED))
    return pl.pallas_call(
        _s_body,
        grid=(_S_GRID,),
        in_specs=[
            pl.BlockSpec((8, EMBED), lambda i: (0, 0)),
            pl.BlockSpec((_S_BLK, EMBED), lambda i: (i, 0)),
        ],
        out_specs=pl.BlockSpec((_S_BLK,), lambda i: (i,)),
        out_shape=jax.ShapeDtypeStruct((VOCAB,), jnp.float32),
    )(w8, table)


_NC = 2
_NS = 16
_NW = _NC * _NS
_ROWS_W = BATCH // _NW
_GRP = 16
_NGRP = _ROWS_W // _GRP

_GSZ = HIST * _GRP


@functools.cache
def _make_sc_pool():
    mesh = plsc.VectorSubcoreMesh(core_axis_name="c", subcore_axis_name="s")

    @functools.partial(
        pl.kernel,
        out_type=jax.ShapeDtypeStruct((BATCH,), jnp.float32),
        mesh=mesh,
        scratch_types=[
            pltpu.VMEM((_GSZ,), jnp.int32),
            pltpu.VMEM((_GSZ,), jnp.float32),
            pltpu.VMEM((_GRP,), jnp.float32),
            pltpu.SemaphoreType.DMA,
        ],
    )
    def _sc_pool(xg_hbm, s_hbm, bv_hbm, out_hbm, idx_v, vals_v, out_v, sem):
        wid = lax.axis_index("s") * _NC + lax.axis_index("c")
        gbase = wid * _NGRP
        pltpu.sync_copy(bv_hbm, out_v)
        bv = out_v[...]

        def group(g, carry):
            ggl = gbase + g
            pltpu.sync_copy(xg_hbm.at[pl.ds(ggl * _GSZ, _GSZ)], idx_v)
            pltpu.async_copy(s_hbm.at[idx_v], vals_v, sem).wait()

            def accum(l, a):
                return a + vals_v[pl.ds(l * _GRP, _GRP)]

            acc = lax.fori_loop(0, HIST, accum, bv)
            out_v[...] = 1.0 / (1.0 + jnp.exp(-acc))
            pltpu.sync_copy(out_v, out_hbm.at[pl.ds(ggl * _GRP, _GRP)])
            return carry

        lax.fori_loop(0, _NGRP, group, 0)

    return _sc_pool


def kernel(x, table, W, b):
    w_scaled = (W / HIST).astype(jnp.float32)
    s = _compute_s(table, w_scaled)
    xg = (
        x.astype(jnp.int32)
        .reshape(BATCH // _GRP, _GRP, HIST)
        .swapaxes(1, 2)
        .reshape(BATCH // _GRP * _GSZ)
    )
    bv = jnp.broadcast_to(b.astype(jnp.float32), (_GRP,))
    return _make_sc_pool()(xg, s, bv)


# speedup vs baseline: 15.5804x; 1.0025x over previous
import functools

import jax
import jax.numpy as jnp
from jax import lax
from jax.experimental import pallas as pl
from jax.experimental.pallas import tpu as pltpu
from jax.experimental.pallas import tpu_sc as plsc

VOCAB = 1000000
EMBED = 32
BATCH = 16384
HIST = 200


_S_BLK = 32768
_S_GRID = (VOCAB + _S_BLK - 1) // _S_BLK


def _s_body(w_ref, t_ref, o_ref):
    o = lax.dot_general(
        w_ref[...], t_ref[...],
        dimension_numbers=(((1,), (1,)), ((), ())),
        preferred_element_type=jnp.float32,
    )
    o_ref[...] = o[0]


def _compute_s(table, w_scaled):
    w8 = jnp.broadcast_to(w_scaled, (8, EMBED))
    return pl.pallas_call(
        _s_body,
        grid=(_S_GRID,),
        in_specs=[
            pl.BlockSpec((8, EMBED), lambda i: (0, 0)),
            pl.BlockSpec((_S_BLK, EMBED), lambda i: (i, 0)),
        ],
        out_specs=pl.BlockSpec((_S_BLK,), lambda i: (i,)),
        out_shape=jax.ShapeDtypeStruct((VOCAB,), jnp.float32),
    )(w8, table)


_T_GBLK = 8


def _t_body(x_ref, o_ref):
    blk = x_ref[...]
    o_ref[...] = (
        blk.reshape(_T_GBLK, 16, HIST).swapaxes(1, 2).reshape(_T_GBLK, 16 * HIST)
    )


def _transpose_groups(x):
    ngrp = BATCH // 16
    return pl.pallas_call(
        _t_body,
        grid=(ngrp // _T_GBLK,),
        in_specs=[pl.BlockSpec((_T_GBLK * 16, HIST), lambda i: (i, 0))],
        out_specs=pl.BlockSpec((_T_GBLK, 16 * HIST), lambda i: (i, 0)),
        out_shape=jax.ShapeDtypeStruct((ngrp, 16 * HIST), jnp.int32),
    )(x)


_NC = 2
_NS = 16
_NW = _NC * _NS
_ROWS_W = BATCH // _NW
_GRP = 16
_NGRP = _ROWS_W // _GRP

_GSZ = HIST * _GRP


@functools.cache
def _make_sc_pool():
    mesh = plsc.VectorSubcoreMesh(core_axis_name="c", subcore_axis_name="s")

    @functools.partial(
        pl.kernel,
        out_type=jax.ShapeDtypeStruct((BATCH,), jnp.float32),
        mesh=mesh,
        scratch_types=[
            pltpu.VMEM((_GSZ,), jnp.int32),
            pltpu.VMEM((_GSZ,), jnp.float32),
            pltpu.VMEM((_GRP,), jnp.float32),
            pltpu.SemaphoreType.DMA,
        ],
    )
    def _sc_pool(xg_hbm, s_hbm, bv_hbm, out_hbm, idx_v, vals_v, out_v, sem):
        wid = lax.axis_index("s") * _NC + lax.axis_index("c")
        gbase = wid * _NGRP
        pltpu.sync_copy(bv_hbm, out_v)
        bv = out_v[...]

        def group(g, carry):
            ggl = gbase + g
            pltpu.sync_copy(xg_hbm.at[ggl], idx_v)
            pltpu.async_copy(s_hbm.at[idx_v], vals_v, sem).wait()

            def accum(l, a):
                return a + vals_v[pl.ds(l * _GRP, _GRP)]

            acc = lax.fori_loop(0, HIST, accum, bv)
            out_v[...] = 1.0 / (1.0 + jnp.exp(-acc))
            pltpu.sync_copy(out_v, out_hbm.at[pl.ds(ggl * _GRP, _GRP)])
            return carry

        lax.fori_loop(0, _NGRP, group, 0)

    return _sc_pool


def kernel(x, table, W, b):
    w_scaled = (W / HIST).astype(jnp.float32)
    s = _compute_s(table, w_scaled)
    xg = _transpose_groups(x.astype(jnp.int32))
    bv = jnp.broadcast_to(b.astype(jnp.float32), (_GRP,))
    return _make_sc_pool()(xg, s, bv)


# speedup vs baseline: 33.5502x; 2.1534x over previous
import functools

import jax
import jax.numpy as jnp
from jax import lax
from jax.experimental import pallas as pl
from jax.experimental.pallas import tpu as pltpu
from jax.experimental.pallas import tpu_sc as plsc

VOCAB = 1000000
EMBED = 32
BATCH = 16384
HIST = 200


_S_BLK = 32768
_S_GRID = (VOCAB + _S_BLK - 1) // _S_BLK


def _s_body(w_ref, t_ref, o_ref):
    o = lax.dot_general(
        w_ref[...], t_ref[...],
        dimension_numbers=(((1,), (0,)), ((), ())),
        preferred_element_type=jnp.float32,
    )
    o_ref[...] = o[0]


def _compute_s(table_t, w_scaled):
    w8 = jnp.broadcast_to(w_scaled, (8, EMBED))
    return pl.pallas_call(
        _s_body,
        grid=(_S_GRID,),
        in_specs=[
            pl.BlockSpec((8, EMBED), lambda i: (0, 0)),
            pl.BlockSpec((EMBED, _S_BLK), lambda i: (0, i)),
        ],
        out_specs=pl.BlockSpec((_S_BLK,), lambda i: (i,)),
        out_shape=jax.ShapeDtypeStruct((VOCAB,), jnp.float32),
    )(w8, table_t)


_T_GBLK = 8


def _t_body(x_ref, o_ref):
    blk = x_ref[...]
    o_ref[...] = (
        blk.reshape(HIST, _T_GBLK, 16).transpose(1, 0, 2).reshape(_T_GBLK, 16 * HIST)
    )


def _transpose_groups(x_t):
    ngrp = BATCH // 16
    return pl.pallas_call(
        _t_body,
        grid=(ngrp // _T_GBLK,),
        in_specs=[pl.BlockSpec((HIST, _T_GBLK * 16), lambda i: (0, i))],
        out_specs=pl.BlockSpec((_T_GBLK, 16 * HIST), lambda i: (i, 0)),
        out_shape=jax.ShapeDtypeStruct((ngrp, 16 * HIST), jnp.int32),
    )(x_t)


_NC = 2
_NS = 16
_NW = _NC * _NS
_ROWS_W = BATCH // _NW
_GRP = 16
_NGRP = _ROWS_W // _GRP

_GSZ = HIST * _GRP


@functools.cache
def _make_sc_pool():
    mesh = plsc.VectorSubcoreMesh(core_axis_name="c", subcore_axis_name="s")

    @functools.partial(
        pl.kernel,
        out_type=jax.ShapeDtypeStruct((BATCH,), jnp.float32),
        mesh=mesh,
        scratch_types=[
            pltpu.VMEM((_GSZ,), jnp.int32),
            pltpu.VMEM((_GSZ,), jnp.float32),
            pltpu.VMEM((_GRP,), jnp.float32),
            pltpu.SemaphoreType.DMA,
        ],
    )
    def _sc_pool(xg_hbm, s_hbm, bv_hbm, out_hbm, idx_v, vals_v, out_v, sem):
        wid = lax.axis_index("s") * _NC + lax.axis_index("c")
        gbase = wid * _NGRP
        pltpu.sync_copy(bv_hbm, out_v)
        bv = out_v[...]

        def group(g, carry):
            ggl = gbase + g
            pltpu.sync_copy(xg_hbm.at[ggl], idx_v)
            pltpu.async_copy(s_hbm.at[idx_v], vals_v, sem).wait()

            def accum(l, a):
                return a + vals_v[pl.ds(l * _GRP, _GRP)]

            acc = lax.fori_loop(0, HIST, accum, bv)
            out_v[...] = 1.0 / (1.0 + jnp.exp(-acc))
            pltpu.sync_copy(out_v, out_hbm.at[pl.ds(ggl * _GRP, _GRP)])
            return carry

        lax.fori_loop(0, _NGRP, group, 0)

    return _sc_pool


def kernel(x, table, W, b):
    w_scaled = (W / HIST).astype(jnp.float32)
    s = _compute_s(table.T, w_scaled)
    xg = _transpose_groups(x.astype(jnp.int32).T)
    bv = jnp.broadcast_to(b.astype(jnp.float32), (_GRP,))
    return _make_sc_pool()(xg, s, bv)
